# Initial kernel scaffold; baseline (speedup 1.0000x reference)
import jax
import jax.numpy as jnp
from jax.experimental import pallas as pl


def kernel(x, edge_index, W, b):
    raise NotImplementedError("write your pallas kernel here")


# speedup vs baseline: 2.5625x; 2.5625x over previous
import jax
import jax.numpy as jnp
from jax import lax
from jax.experimental import pallas as pl
from jax.experimental.pallas import tpu as pltpu
from jax.experimental.pallas import tpu_sc as plsc

N_NODES = 10000
D = 128
N_EDGES = 320000

NC = 2
NS = 16
K = 128
CPS = 160
E_PAD = NS * CPS * K
M_PAD = 10240
HR = M_PAD // NC
HRP = HR + 8
RPT = HR // NS
SCHUNKS = ((0, 160), (160, 160))
DROWS = M_PAD // K


def _sc_body(x_hbm, srcm, dstm, acc_out,
             src_v, dst_v, dloc_v, rows_v, sbuf_v, hist_v, rid_v, acc_sh,
             deg_sh, sem):
    c = lax.axis_index("c")
    s = lax.axis_index("s")
    base_c = c * HR

    pltpu.sync_copy(srcm.at[pl.ds(s * CPS, CPS)], src_v)
    pltpu.sync_copy(dstm.at[pl.ds(s * CPS, CPS)], dst_v)

    def zero_hist(i, carry):
        for k in range(D // 16):
            hist_v[i, pl.ds(k * 16, 16)] = jnp.zeros((16,), jnp.float32)
        return carry
    lax.fori_loop(0, DROWS, zero_hist, None)

    def fill_rid(i, carry):
        rid_v[pl.ds(i * 16, 16)] = lax.iota(jnp.int32, 16) + i * 16
        return carry
    lax.fori_loop(0, DROWS // 16, fill_rid, None)

    def zero_sbuf(i, carry):
        for k in range(D // 16):
            sbuf_v[i, pl.ds(k * 16, 16)] = jnp.zeros((16,), jnp.float32)
        return carry
    lax.fori_loop(0, 160, zero_sbuf, None)

    for off, ln in SCHUNKS:
        pltpu.sync_copy(sbuf_v.at[pl.ds(0, ln)],
                        acc_sh.at[pl.ds(s * RPT + off, ln)])

    @pl.when(s == 0)
    def _zero_tail():
        pltpu.sync_copy(sbuf_v.at[pl.ds(0, 8)], acc_sh.at[pl.ds(HR, 8)])
        pltpu.sync_copy(sbuf_v.at[pl.ds(0, DROWS)], deg_sh)
    plsc.subcore_barrier()

    def step(j, carry):
        cp = pltpu.async_copy(x_hbm.at[src_v.at[j]], rows_v, sem)
        for k in range(K // 16):
            sl = pl.ds(k * 16, 16)
            d = dst_v[j, sl]
            cnt, last = plsc.scan_count(d)
            plsc.addupdate_scatter(hist_v,
                                   [lax.shift_right_logical(d, 7),
                                    lax.bitwise_and(d, 127)],
                                   cnt.astype(jnp.float32), mask=last)
            loc = d - base_c
            inr = jnp.logical_and(loc >= 0, loc < HR)
            dloc_v[sl] = jnp.where(inr, loc, HR)
        cp.wait()
        pltpu.sync_copy(rows_v, acc_sh.at[dloc_v], add=True)
        return carry
    lax.fori_loop(0, CPS, step, None)

    pltpu.sync_copy(hist_v, deg_sh.at[rid_v], add=True)
    plsc.subcore_barrier()
    pltpu.sync_copy(deg_sh, hist_v)

    for off, ln in SCHUNKS:
        pltpu.sync_copy(acc_sh.at[pl.ds(s * RPT + off, ln)],
                        sbuf_v.at[pl.ds(0, ln)])

        def scale_row(r, carry, off=off):
            flat = base_c + s * RPT + off + r
            row = jnp.full((16,), lax.shift_right_logical(flat, 7), jnp.int32)
            lane = jnp.full((16,), lax.bitwise_and(flat, 127), jnp.int32)
            dv = plsc.load_gather(hist_v, [row, lane])
            rec = 1.0 / jnp.maximum(dv, 1.0)
            for k in range(D // 16):
                sl = pl.ds(k * 16, 16)
                sbuf_v[r, sl] = sbuf_v[r, sl] * rec
            return carry
        lax.fori_loop(0, ln, scale_row, None)
        pltpu.sync_copy(sbuf_v.at[pl.ds(0, ln)],
                        acc_out.at[c, pl.ds(s * RPT + off, ln)])


def _sc_aggregate(x, srcm, dstm):
    mesh = plsc.VectorSubcoreMesh(core_axis_name="c", subcore_axis_name="s")
    return pl.kernel(
        _sc_body,
        out_type=jax.ShapeDtypeStruct((NC, HR, D), jnp.float32),
        mesh=mesh,
        scratch_types=[
            pltpu.VMEM((CPS, K), jnp.int32),
            pltpu.VMEM((CPS, K), jnp.int32),
            pltpu.VMEM((K,), jnp.int32),
            pltpu.VMEM((K, D), jnp.float32),
            pltpu.VMEM((160, D), jnp.float32),
            pltpu.VMEM((DROWS, D), jnp.float32),
            pltpu.VMEM((DROWS,), jnp.int32),
            pltpu.VMEM_SHARED((HRP, D), jnp.float32),
            pltpu.VMEM_SHARED((DROWS, D), jnp.float32),
            pltpu.SemaphoreType.DMA,
        ],
        compiler_params=pltpu.CompilerParams(
            use_tc_tiling_on_sc=False, needs_layout_passes=False),
    )(x, srcm, dstm)


def _tc_body(x_ref, p_ref, w_ref, b_ref, o_ref):
    h = (jnp.dot(x_ref[...], w_ref[0:D, :], preferred_element_type=jnp.float32)
         + jnp.dot(p_ref[0], w_ref[D:, :], preferred_element_type=jnp.float32)
         + b_ref[...])
    o_ref[...] = jnp.maximum(h, 0.0)


def _tc_linear(xp, acc, W, b):
    bm = 640
    nb = HR // bm
    return pl.pallas_call(
        _tc_body,
        grid=(M_PAD // bm,),
        in_specs=[
            pl.BlockSpec((bm, D), lambda i: (i, 0)),
            pl.BlockSpec((1, bm, D), lambda i: (i // nb, i % nb, 0)),
            pl.BlockSpec((2 * D, D), lambda i: (0, 0)),
            pl.BlockSpec((1, D), lambda i: (0, 0)),
        ],
        out_specs=pl.BlockSpec((bm, D), lambda i: (i, 0)),
        out_shape=jax.ShapeDtypeStruct((M_PAD, D), jnp.float32),
    )(xp, acc, W, b.reshape(1, D))


def kernel(x, edge_index, W, b):
    src = edge_index[0].astype(jnp.int32)
    dst = edge_index[1].astype(jnp.int32)
    pad = E_PAD - N_EDGES
    srcm = jnp.concatenate([src, jnp.zeros((pad,), jnp.int32)]).reshape(-1, K)
    dstm = jnp.concatenate(
        [dst, jnp.full((pad,), N_NODES, jnp.int32)]).reshape(-1, K)
    xp = jnp.concatenate([x, jnp.zeros((M_PAD - N_NODES, D), x.dtype)])
    xp, srcm, dstm = lax.optimization_barrier((xp, srcm, dstm))
    acc = _sc_aggregate(xp, srcm, dstm)
    return _tc_linear(xp, acc, W, b)[:N_NODES]
